# Initial kernel scaffold; baseline (speedup 1.0000x reference)
import jax
import jax.numpy as jnp
from jax.experimental import pallas as pl


def kernel(input_ids, emb_table):
    raise NotImplementedError("write your pallas kernel here")


# speedup vs baseline: 1.2125x; 1.2125x over previous
import functools

import jax
import jax.numpy as jnp
from jax import lax
from jax.experimental import pallas as pl
from jax.experimental.pallas import tpu as pltpu
from jax.experimental.pallas import tpu_sc as plsc

D_MODEL = 768
VOCAB = 100000
BATCH = 4
SEQ = 8192
SCALE = D_MODEL ** 0.5

_INFO = plsc.get_sparse_core_info()
_NC = _INFO.num_cores
_NS = _INFO.num_subcores
_L = _INFO.num_lanes
_NW = _NC * _NS

_B_TOT = BATCH * SEQ
_PER_W = _B_TOT // _NW
_C = 64
_NCH = _PER_W // _C

_mesh = plsc.VectorSubcoreMesh(core_axis_name="c", subcore_axis_name="s")


@functools.partial(
    pl.kernel,
    mesh=_mesh,
    out_type=jax.ShapeDtypeStruct((_B_TOT, D_MODEL), jnp.float32),
    scratch_types=[
        pltpu.VMEM((_PER_W,), jnp.int32),
        pltpu.VMEM((_C, D_MODEL), jnp.float32),
        pltpu.SemaphoreType.DMA,
    ],
)
def _gather_scale(ids_hbm, table_hbm, out_hbm, idx_v, rows_v, sem):
    wid = lax.axis_index("s") * _NC + lax.axis_index("c")
    base = wid * _PER_W
    pltpu.sync_copy(ids_hbm.at[pl.ds(base, _PER_W)], idx_v)

    def chunk_body(g, carry):
        pltpu.async_copy(
            table_hbm.at[idx_v.at[pl.ds(g * _C, _C)]], rows_v, sem
        ).wait()

        def row_body(r, c2):
            for j in range(D_MODEL // _L):
                sl = pl.ds(j * _L, _L)
                rows_v[r, sl] = rows_v[r, sl] * SCALE
            return c2

        lax.fori_loop(0, _C, row_body, 0)
        pltpu.sync_copy(rows_v, out_hbm.at[pl.ds(base + g * _C, _C)])
        return carry

    lax.fori_loop(0, _NCH, chunk_body, 0)


def kernel(input_ids, emb_table):
    ids_flat = input_ids.reshape(-1).astype(jnp.int32)
    out = _gather_scale(ids_flat, emb_table)
    return out.reshape(BATCH, SEQ, D_MODEL)


# speedup vs baseline: 1.5572x; 1.2843x over previous
import functools

import jax
import jax.numpy as jnp
from jax import lax
from jax.experimental import pallas as pl
from jax.experimental.pallas import tpu as pltpu
from jax.experimental.pallas import tpu_sc as plsc

D_MODEL = 768
VOCAB = 100000
BATCH = 4
SEQ = 8192
SCALE = D_MODEL ** 0.5

_INFO = plsc.get_sparse_core_info()
_NC = _INFO.num_cores
_NS = _INFO.num_subcores
_L = _INFO.num_lanes
_NW = _NC * _NS

_B_TOT = BATCH * SEQ
_PER_W = _B_TOT // _NW
_C = 64
_NCH = _PER_W // _C

_mesh = plsc.VectorSubcoreMesh(core_axis_name="c", subcore_axis_name="s")


@functools.partial(
    pl.kernel,
    mesh=_mesh,
    out_type=jax.ShapeDtypeStruct((_B_TOT, D_MODEL), jnp.float32),
    scratch_types=[
        pltpu.VMEM((_PER_W,), jnp.int32),
        pltpu.VMEM((2, _C, D_MODEL), jnp.float32),
        pltpu.SemaphoreType.DMA,
        pltpu.SemaphoreType.DMA,
        pltpu.SemaphoreType.DMA,
        pltpu.SemaphoreType.DMA,
    ],
)
def _gather_scale(ids_hbm, table_hbm, out_hbm, idx_v, rows_v, g0, g1, s0, s1):
    gsem = (g0, g1)
    ssem = (s0, s1)
    wid = lax.axis_index("s") * _NC + lax.axis_index("c")
    base = wid * _PER_W
    pltpu.sync_copy(ids_hbm.at[pl.ds(base, _PER_W)], idx_v)

    def start_gather(g, b):
        return pltpu.async_copy(
            table_hbm.at[idx_v.at[pl.ds(g * _C, _C)]], rows_v.at[b], gsem[b]
        )

    gather_h = [start_gather(0, 0), None]
    store_h = [None, None]
    for g in range(_NCH):
        b = g % 2
        nb = (g + 1) % 2
        if g + 1 < _NCH:
            if store_h[nb] is not None:
                store_h[nb].wait()
            gather_h[nb] = start_gather(g + 1, nb)
        gather_h[b].wait()

        def row_body(r, carry, b=b):
            for j in range(D_MODEL // _L):
                sl = pl.ds(j * _L, _L)
                rows_v[b, r, sl] = rows_v[b, r, sl] * SCALE
            return carry

        lax.fori_loop(0, _C, row_body, 0)
        store_h[b] = pltpu.async_copy(
            rows_v.at[b], out_hbm.at[pl.ds(base + g * _C, _C)], ssem[b]
        )
    store_h[0].wait()
    store_h[1].wait()


def kernel(input_ids, emb_table):
    ids_flat = input_ids.reshape(-1).astype(jnp.int32)
    out = _gather_scale(ids_flat, emb_table)
    return out.reshape(BATCH, SEQ, D_MODEL)
